# Initial kernel scaffold; baseline (speedup 1.0000x reference)
import jax
import jax.numpy as jnp
from jax.experimental import pallas as pl


def kernel(x, table, Wp, Wt0, bt0, Wg0, bg0, Wt1, bt1, Wg1, bg1):
    raise NotImplementedError("write your pallas kernel here")


# speedup vs baseline: 6.7509x; 6.7509x over previous
import functools

import jax
import jax.numpy as jnp
from jax import lax
from jax.experimental import pallas as pl
from jax.experimental.pallas import tpu as pltpu
from jax.experimental.pallas import tpu_sc as plsc

VOCAB, EDIM, HID = 100000, 64, 128


_ROWS_PER_BLK = 1000


def _mlp_body(tab, Wp, Wt0, bt0, Wg0, bg0, Wt1, bt1, Wg1, bg1, out):
    h = jnp.dot(tab[...], Wp[...], preferred_element_type=jnp.float32)
    for Wt, bt, Wg, bg in ((Wt0, bt0, Wg0, bg0), (Wt1, bt1, Wg1, bg1)):
        g = jax.nn.sigmoid(jnp.dot(h, Wg[...], preferred_element_type=jnp.float32) + bg[...])
        t = jnp.maximum(jnp.dot(h, Wt[...], preferred_element_type=jnp.float32) + bt[...], 0.0)
        h = g * t + (1.0 - g) * h
    out[...] = h


def _fuse_table(table, Wp, Wt0, bt0, Wg0, bg0, Wt1, bt1, Wg1, bg1):
    n_blk = VOCAB // _ROWS_PER_BLK
    full = lambda shape: pl.BlockSpec(shape, lambda i: (0, 0))
    return pl.pallas_call(
        _mlp_body,
        grid=(n_blk,),
        in_specs=[
            pl.BlockSpec((_ROWS_PER_BLK, EDIM), lambda i: (i, 0)),
            full((EDIM, HID)),
            full((HID, HID)), full((1, HID)),
            full((HID, HID)), full((1, HID)),
            full((HID, HID)), full((1, HID)),
            full((HID, HID)), full((1, HID)),
        ],
        out_specs=pl.BlockSpec((_ROWS_PER_BLK, HID), lambda i: (i, 0)),
        out_shape=jax.ShapeDtypeStruct((VOCAB, HID), jnp.float32),
    )(table, Wp,
      Wt0, bt0.reshape(1, HID), Wg0, bg0.reshape(1, HID),
      Wt1, bt1.reshape(1, HID), Wg1, bg1.reshape(1, HID))


_CHUNK = 128


def _make_sc_gather(B):
    info = plsc.get_sparse_core_info()
    NC, NS = info.num_cores, info.num_subcores
    NW = NC * NS
    assert B % (NW * _CHUNK) == 0
    b_per_w = B // NW
    n_chunks = b_per_w // _CHUNK
    mesh = plsc.VectorSubcoreMesh(core_axis_name="c", subcore_axis_name="s")

    @functools.partial(
        pl.kernel,
        mesh=mesh,
        out_type=jax.ShapeDtypeStruct((B, HID), jnp.float32),
        scratch_types=[
            pltpu.VMEM((_CHUNK,), jnp.int32),
            pltpu.VMEM((_CHUNK, HID), jnp.float32),
            pltpu.SemaphoreType.DMA,
        ],
    )
    def sc_gather(ftab_hbm, idx_hbm, out_hbm, idx_v, rows_v, sem):
        wid = lax.axis_index("s") * NC + lax.axis_index("c")
        base = wid * b_per_w

        def chunk(j, carry):
            off = base + j * _CHUNK
            pltpu.sync_copy(idx_hbm.at[pl.ds(off, _CHUNK)], idx_v)
            pltpu.async_copy(ftab_hbm.at[idx_v], rows_v, sem).wait()
            pltpu.sync_copy(rows_v, out_hbm.at[pl.ds(off, _CHUNK)])
            return carry

        lax.fori_loop(0, n_chunks, chunk, 0)

    return sc_gather


def kernel(x, table, Wp, Wt0, bt0, Wg0, bg0, Wt1, bt1, Wg1, bg1):
    B, L = x.shape
    ftab = _fuse_table(table, Wp, Wt0, bt0, Wg0, bg0, Wt1, bt1, Wg1, bg1)
    out = _make_sc_gather(B * L)(ftab, x.reshape(-1))
    return out.reshape(B, L, HID)


# speedup vs baseline: 10.3810x; 1.5377x over previous
import functools

import jax
import jax.numpy as jnp
from jax import lax
from jax.experimental import pallas as pl
from jax.experimental.pallas import tpu as pltpu
from jax.experimental.pallas import tpu_sc as plsc

VOCAB, EDIM, HID = 100000, 64, 128


_ROWS_PER_BLK = 1000


def _mlp_body(tab, Wp, Wt0, bt0, Wg0, bg0, Wt1, bt1, Wg1, bg1, out):
    h = jnp.dot(tab[...], Wp[...], preferred_element_type=jnp.float32)
    for Wt, bt, Wg, bg in ((Wt0, bt0, Wg0, bg0), (Wt1, bt1, Wg1, bg1)):
        g = jax.nn.sigmoid(jnp.dot(h, Wg[...], preferred_element_type=jnp.float32) + bg[...])
        t = jnp.maximum(jnp.dot(h, Wt[...], preferred_element_type=jnp.float32) + bt[...], 0.0)
        h = g * t + (1.0 - g) * h
    out[...] = h


def _fuse_table(table, Wp, Wt0, bt0, Wg0, bg0, Wt1, bt1, Wg1, bg1):
    n_blk = VOCAB // _ROWS_PER_BLK
    full = lambda shape: pl.BlockSpec(shape, lambda i: (0, 0))
    return pl.pallas_call(
        _mlp_body,
        grid=(n_blk,),
        in_specs=[
            pl.BlockSpec((_ROWS_PER_BLK, EDIM), lambda i: (i, 0)),
            full((EDIM, HID)),
            full((HID, HID)), full((1, HID)),
            full((HID, HID)), full((1, HID)),
            full((HID, HID)), full((1, HID)),
            full((HID, HID)), full((1, HID)),
        ],
        out_specs=pl.BlockSpec((_ROWS_PER_BLK, HID), lambda i: (i, 0)),
        out_shape=jax.ShapeDtypeStruct((VOCAB, HID), jnp.float32),
    )(table, Wp,
      Wt0, bt0.reshape(1, HID), Wg0, bg0.reshape(1, HID),
      Wt1, bt1.reshape(1, HID), Wg1, bg1.reshape(1, HID))


_CHUNK = 128
_NBUF = 4


def _make_sc_gather(B):
    info = plsc.get_sparse_core_info()
    NC, NS = info.num_cores, info.num_subcores
    NW = NC * NS
    assert B % (NW * _CHUNK * _NBUF) == 0
    b_per_w = B // NW
    n_chunks = b_per_w // _CHUNK
    n_groups = n_chunks // _NBUF
    assert n_groups >= 3
    mesh = plsc.VectorSubcoreMesh(core_axis_name="c", subcore_axis_name="s")

    @functools.partial(
        pl.kernel,
        mesh=mesh,
        out_type=jax.ShapeDtypeStruct((B, HID), jnp.float32),
        scratch_types=[
            pltpu.VMEM((n_chunks, _CHUNK), jnp.int32),
            [pltpu.VMEM((_CHUNK, HID), jnp.float32) for _ in range(_NBUF)],
            [pltpu.SemaphoreType.DMA for _ in range(_NBUF)],
            [pltpu.SemaphoreType.DMA for _ in range(_NBUF)],
        ],
    )
    def sc_gather(ftab_hbm, idx_hbm, out_hbm, idx_v, bufs, sem_g, sem_w):
        wid = lax.axis_index("s") * NC + lax.axis_index("c")
        base = wid * b_per_w

        pltpu.sync_copy(idx_hbm.at[pl.ds(wid * n_chunks, n_chunks)], idx_v)

        def fire_g(j, b):
            pltpu.async_copy(ftab_hbm.at[idx_v.at[j]], bufs[b], sem_g[b])

        def wait_g(b):
            pltpu.make_async_copy(ftab_hbm.at[idx_v.at[0]], bufs[b], sem_g[b]).wait()

        def fire_w(j, b):
            pltpu.async_copy(bufs[b], out_hbm.at[pl.ds(base + j * _CHUNK, _CHUNK)], sem_w[b])

        def wait_w(b):
            pltpu.make_async_copy(bufs[b], out_hbm.at[pl.ds(base, _CHUNK)], sem_w[b]).wait()

        fire_g(0, 0)
        fire_g(1, 1)
        for b in range(_NBUF):
            if b >= 2:
                wait_w((b + 2) % _NBUF)
            fire_g(b + 2, (b + 2) % _NBUF)
            wait_g(b)
            fire_w(b, b)

        @pl.loop(1, n_groups - 1)
        def _(g):
            j0 = g * _NBUF
            for b in range(_NBUF):
                bn = (b + 2) % _NBUF
                wait_w(bn)
                fire_g(j0 + b + 2, bn)
                wait_g(b)
                fire_w(j0 + b, b)

        m = n_chunks - _NBUF
        for b in range(_NBUF):
            if b < 2:
                wait_w((b + 2) % _NBUF)
                fire_g(m + b + 2, (b + 2) % _NBUF)
            wait_g(b)
            fire_w(m + b, b)
        for b in range(_NBUF):
            wait_w(b)

    return sc_gather


def kernel(x, table, Wp, Wt0, bt0, Wg0, bg0, Wt1, bt1, Wg1, bg1):
    B, L = x.shape
    ftab = _fuse_table(table, Wp, Wt0, bt0, Wg0, bg0, Wt1, bt1, Wg1, bg1)
    idx2d = x.reshape(B * L // _CHUNK, _CHUNK)
    out = _make_sc_gather(B * L)(ftab, idx2d)
    return out.reshape(B, L, HID)


# speedup vs baseline: 11.3006x; 1.0886x over previous
import functools

import jax
import jax.numpy as jnp
from jax import lax
from jax.experimental import pallas as pl
from jax.experimental.pallas import tpu as pltpu
from jax.experimental.pallas import tpu_sc as plsc

VOCAB, EDIM, HID = 100000, 64, 128


_ROWS_PER_BLK = 2000


def _mlp_body(tab, Wp, Wt0, bt0, Wg0, bg0, Wt1, bt1, Wg1, bg1, out):
    h = jnp.dot(tab[...], Wp[...], preferred_element_type=jnp.float32)
    for Wt, bt, Wg, bg in ((Wt0, bt0, Wg0, bg0), (Wt1, bt1, Wg1, bg1)):
        g = jax.nn.sigmoid(jnp.dot(h, Wg[...], preferred_element_type=jnp.float32) + bg[...])
        t = jnp.maximum(jnp.dot(h, Wt[...], preferred_element_type=jnp.float32) + bt[...], 0.0)
        h = g * t + (1.0 - g) * h
    out[...] = h


def _fuse_table(table, Wp, Wt0, bt0, Wg0, bg0, Wt1, bt1, Wg1, bg1):
    n_blk = VOCAB // _ROWS_PER_BLK
    full = lambda shape: pl.BlockSpec(shape, lambda i: (0, 0))
    return pl.pallas_call(
        _mlp_body,
        grid=(n_blk,),
        in_specs=[
            pl.BlockSpec((_ROWS_PER_BLK, EDIM), lambda i: (i, 0)),
            full((EDIM, HID)),
            full((HID, HID)), full((1, HID)),
            full((HID, HID)), full((1, HID)),
            full((HID, HID)), full((1, HID)),
            full((HID, HID)), full((1, HID)),
        ],
        out_specs=pl.BlockSpec((_ROWS_PER_BLK, HID), lambda i: (i, 0)),
        out_shape=jax.ShapeDtypeStruct((VOCAB, HID), jnp.float32),
    )(table, Wp,
      Wt0, bt0.reshape(1, HID), Wg0, bg0.reshape(1, HID),
      Wt1, bt1.reshape(1, HID), Wg1, bg1.reshape(1, HID))


_CHUNK = 128
_NBUF = 4


def _make_sc_gather(B):
    info = plsc.get_sparse_core_info()
    NC, NS = info.num_cores, info.num_subcores
    NW = NC * NS
    assert B % (NW * _CHUNK * _NBUF) == 0
    b_per_w = B // NW
    n_chunks = b_per_w // _CHUNK
    n_groups = n_chunks // _NBUF
    assert n_groups >= 3
    mesh = plsc.VectorSubcoreMesh(core_axis_name="c", subcore_axis_name="s")

    @functools.partial(
        pl.kernel,
        mesh=mesh,
        out_type=jax.ShapeDtypeStruct((B, HID), jnp.float32),
        scratch_types=[
            pltpu.VMEM((n_chunks, _CHUNK), jnp.int32),
            [pltpu.VMEM((_CHUNK, HID), jnp.float32) for _ in range(_NBUF)],
            [pltpu.SemaphoreType.DMA for _ in range(_NBUF)],
            [pltpu.SemaphoreType.DMA for _ in range(_NBUF)],
        ],
    )
    def sc_gather(ftab_hbm, idx_hbm, out_hbm, idx_v, bufs, sem_g, sem_w):
        wid = lax.axis_index("s") * NC + lax.axis_index("c")
        base = wid * b_per_w

        pltpu.sync_copy(idx_hbm.at[pl.ds(wid * n_chunks, n_chunks)], idx_v)

        def fire_g(j, b):
            pltpu.async_copy(ftab_hbm.at[idx_v.at[j]], bufs[b], sem_g[b])

        def wait_g(b):
            pltpu.make_async_copy(ftab_hbm.at[idx_v.at[0]], bufs[b], sem_g[b]).wait()

        def fire_w(j, b):
            pltpu.async_copy(bufs[b], out_hbm.at[pl.ds(base + j * _CHUNK, _CHUNK)], sem_w[b])

        def wait_w(b):
            pltpu.make_async_copy(bufs[b], out_hbm.at[pl.ds(base, _CHUNK)], sem_w[b]).wait()

        fire_g(0, 0)
        fire_g(1, 1)
        for b in range(_NBUF):
            if b >= 2:
                wait_w((b + 2) % _NBUF)
            fire_g(b + 2, (b + 2) % _NBUF)
            wait_g(b)
            fire_w(b, b)

        @pl.loop(1, n_groups - 1)
        def _(g):
            j0 = g * _NBUF
            for b in range(_NBUF):
                bn = (b + 2) % _NBUF
                wait_w(bn)
                fire_g(j0 + b + 2, bn)
                wait_g(b)
                fire_w(j0 + b, b)

        m = n_chunks - _NBUF
        for b in range(_NBUF):
            if b < 2:
                wait_w((b + 2) % _NBUF)
                fire_g(m + b + 2, (b + 2) % _NBUF)
            wait_g(b)
            fire_w(m + b, b)
        for b in range(_NBUF):
            wait_w(b)

    return sc_gather


def kernel(x, table, Wp, Wt0, bt0, Wg0, bg0, Wt1, bt1, Wg1, bg1):
    B, L = x.shape
    ftab = _fuse_table(table, Wp, Wt0, bt0, Wg0, bg0, Wt1, bt1, Wg1, bg1)
    idx2d = x.reshape(B * L // _CHUNK, _CHUNK)
    out = _make_sc_gather(B * L)(ftab, idx2d)
    return out.reshape(B, L, HID)


# speedup vs baseline: 11.6739x; 1.0330x over previous
import functools

import jax
import jax.numpy as jnp
from jax import lax
from jax.experimental import pallas as pl
from jax.experimental.pallas import tpu as pltpu
from jax.experimental.pallas import tpu_sc as plsc

VOCAB, EDIM, HID = 100000, 64, 128


_ROWS_PER_BLK = 4000


def _mlp_body(tab, Wp, Wt0, bt0, Wg0, bg0, Wt1, bt1, Wg1, bg1, out):
    h = jnp.dot(tab[...], Wp[...], preferred_element_type=jnp.float32)
    for Wt, bt, Wg, bg in ((Wt0, bt0, Wg0, bg0), (Wt1, bt1, Wg1, bg1)):
        g = jax.nn.sigmoid(jnp.dot(h, Wg[...], preferred_element_type=jnp.float32) + bg[...])
        t = jnp.maximum(jnp.dot(h, Wt[...], preferred_element_type=jnp.float32) + bt[...], 0.0)
        h = g * t + (1.0 - g) * h
    out[...] = h


def _fuse_table(table, Wp, Wt0, bt0, Wg0, bg0, Wt1, bt1, Wg1, bg1):
    n_blk = VOCAB // _ROWS_PER_BLK
    full = lambda shape: pl.BlockSpec(shape, lambda i: (0, 0))
    return pl.pallas_call(
        _mlp_body,
        grid=(n_blk,),
        in_specs=[
            pl.BlockSpec((_ROWS_PER_BLK, EDIM), lambda i: (i, 0)),
            full((EDIM, HID)),
            full((HID, HID)), full((1, HID)),
            full((HID, HID)), full((1, HID)),
            full((HID, HID)), full((1, HID)),
            full((HID, HID)), full((1, HID)),
        ],
        out_specs=pl.BlockSpec((_ROWS_PER_BLK, HID), lambda i: (i, 0)),
        out_shape=jax.ShapeDtypeStruct((VOCAB, HID), jnp.float32),
    )(table, Wp,
      Wt0, bt0.reshape(1, HID), Wg0, bg0.reshape(1, HID),
      Wt1, bt1.reshape(1, HID), Wg1, bg1.reshape(1, HID))


_CHUNK = 128
_NBUF = 4


def _make_sc_gather(B):
    info = plsc.get_sparse_core_info()
    NC, NS = info.num_cores, info.num_subcores
    NW = NC * NS
    assert B % (NW * _CHUNK * _NBUF) == 0
    b_per_w = B // NW
    n_chunks = b_per_w // _CHUNK
    n_groups = n_chunks // _NBUF
    assert n_groups >= 3
    mesh = plsc.VectorSubcoreMesh(core_axis_name="c", subcore_axis_name="s")

    @functools.partial(
        pl.kernel,
        mesh=mesh,
        out_type=jax.ShapeDtypeStruct((B, HID), jnp.float32),
        scratch_types=[
            pltpu.VMEM((n_chunks, _CHUNK), jnp.int32),
            [pltpu.VMEM((_CHUNK, HID), jnp.float32) for _ in range(_NBUF)],
            [pltpu.SemaphoreType.DMA for _ in range(_NBUF)],
            [pltpu.SemaphoreType.DMA for _ in range(_NBUF)],
        ],
    )
    def sc_gather(ftab_hbm, idx_hbm, out_hbm, idx_v, bufs, sem_g, sem_w):
        wid = lax.axis_index("s") * NC + lax.axis_index("c")
        base = wid * b_per_w

        pltpu.sync_copy(idx_hbm.at[pl.ds(wid * n_chunks, n_chunks)], idx_v)

        def fire_g(j, b):
            pltpu.async_copy(ftab_hbm.at[idx_v.at[j]], bufs[b], sem_g[b])

        def wait_g(b):
            pltpu.make_async_copy(ftab_hbm.at[idx_v.at[0]], bufs[b], sem_g[b]).wait()

        def fire_w(j, b):
            pltpu.async_copy(bufs[b], out_hbm.at[pl.ds(base + j * _CHUNK, _CHUNK)], sem_w[b])

        def wait_w(b):
            pltpu.make_async_copy(bufs[b], out_hbm.at[pl.ds(base, _CHUNK)], sem_w[b]).wait()

        fire_g(0, 0)
        fire_g(1, 1)
        for b in range(_NBUF):
            if b >= 2:
                wait_w((b + 2) % _NBUF)
            fire_g(b + 2, (b + 2) % _NBUF)
            wait_g(b)
            fire_w(b, b)

        @pl.loop(1, n_groups - 1)
        def _(g):
            j0 = g * _NBUF
            for b in range(_NBUF):
                bn = (b + 2) % _NBUF
                wait_w(bn)
                fire_g(j0 + b + 2, bn)
                wait_g(b)
                fire_w(j0 + b, b)

        m = n_chunks - _NBUF
        for b in range(_NBUF):
            if b < 2:
                wait_w((b + 2) % _NBUF)
                fire_g(m + b + 2, (b + 2) % _NBUF)
            wait_g(b)
            fire_w(m + b, b)
        for b in range(_NBUF):
            wait_w(b)

    return sc_gather


def kernel(x, table, Wp, Wt0, bt0, Wg0, bg0, Wt1, bt1, Wg1, bg1):
    B, L = x.shape
    ftab = _fuse_table(table, Wp, Wt0, bt0, Wg0, bg0, Wt1, bt1, Wg1, bg1)
    idx2d = x.reshape(B * L // _CHUNK, _CHUNK)
    out = _make_sc_gather(B * L)(ftab, idx2d)
    return out.reshape(B, L, HID)
